# baseline (speedup 1.0000x reference)
import functools

import jax
import jax.numpy as jnp
from jax import lax
from jax.experimental import pallas as pl
from jax.experimental.pallas import tpu as pltpu
from jax.experimental.pallas import tpu_sc as plsc

BATCH = 16384
EMB2 = 64
NC, NS, LANES = 2, 16, 16
NW = NC * NS
BPW = BATCH // NW
CHUNK = 128
NCH = BPW // CHUNK


def _sc_body(qr_hbm, qo_hbm, rel_hbm, ent_hbm, wb_hbm, out_hbm,
             ridx_v, oidx_v, rrows_v, orows_v, out_v, wb_v, sem):
    wid = lax.axis_index("s") * NC + lax.axis_index("c")
    base = wid * BPW

    pltpu.sync_copy(qr_hbm.at[wid], ridx_v)
    pltpu.sync_copy(qo_hbm.at[wid], oidx_v)
    pltpu.sync_copy(wb_hbm, wb_v)

    copies = []
    for j in range(NCH):
        dst = rrows_v.at[pl.ds(j * CHUNK, CHUNK)]
        copies.append(pltpu.async_copy(rel_hbm.at[ridx_v.at[j]], dst, sem))
        dst = orows_v.at[pl.ds(j * CHUNK, CHUNK)]
        copies.append(pltpu.async_copy(ent_hbm.at[oidx_v.at[j]], dst, sem))
    for c in copies:
        c.wait()

    w_vecs = [wb_v[pl.ds(k * LANES, LANES)] for k in range(9)]
    w_rel = [w_vecs[d // LANES][d % LANES] for d in range(EMB2)]
    w_obj = [w_vecs[4 + d // LANES][d % LANES] for d in range(EMB2)]
    bias = w_vecs[8][0]

    def group(g, carry):
        rows = lax.iota(jnp.int32, LANES) + g * LANES
        acc = jnp.full((LANES,), bias, jnp.float32)
        for d in range(EMB2):
            col = jnp.full((LANES,), d, jnp.int32)
            acc = acc + plsc.load_gather(rrows_v, [rows, col]) * w_rel[d]
            acc = acc + plsc.load_gather(orows_v, [rows, col]) * w_obj[d]
        out_v[pl.ds(g * LANES, LANES)] = acc
        return carry

    lax.fori_loop(0, BPW // LANES, group, 0)
    pltpu.sync_copy(out_v, out_hbm.at[pl.ds(base, BPW)])


@jax.jit
def _classify(qr, qo, rel_table, ent_table, wb):
    mesh = plsc.VectorSubcoreMesh(core_axis_name="c", subcore_axis_name="s")
    run = pl.kernel(
        _sc_body,
        out_type=jax.ShapeDtypeStruct((BATCH,), jnp.float32),
        mesh=mesh,
        compiler_params=pltpu.CompilerParams(needs_layout_passes=False, use_tc_tiling_on_sc=False),
        scratch_types=[
            pltpu.VMEM((NCH, CHUNK), jnp.int32),
            pltpu.VMEM((NCH, CHUNK), jnp.int32),
            pltpu.VMEM((BPW, EMB2), jnp.float32),
            pltpu.VMEM((BPW, EMB2), jnp.float32),
            pltpu.VMEM((BPW,), jnp.float32),
            pltpu.VMEM((9 * LANES,), jnp.float32),
            pltpu.SemaphoreType.DMA,
        ],
    )
    return run(qr, qo, rel_table, ent_table, wb)


def kernel(query_relation, query_object, relation_table, entity_table, W, b):
    qr = query_relation.astype(jnp.int32).reshape(NW, NCH, CHUNK)
    qo = query_object.astype(jnp.int32).reshape(NW, NCH, CHUNK)
    wb = jnp.concatenate(
        [W.reshape(4 * 32), b, jnp.zeros((15,), jnp.float32)])
    out = _classify(qr, qo, relation_table, entity_table, wb)
    return out.reshape(BATCH, 1)


# speedup vs baseline: 1.7132x; 1.7132x over previous
import jax
import jax.numpy as jnp
from jax import lax
from jax.experimental import pallas as pl
from jax.experimental.pallas import tpu as pltpu
from jax.experimental.pallas import tpu_sc as plsc

BATCH = 16384
EMB2 = 64
NC, NS, LANES = 2, 16, 16
NW = NC * NS
BPW = BATCH // NW
REL_PAD = 1024


def _score_body(rel_ref, w_ref, b_ref, o_ref):
    w_rel = w_ref[0:EMB2, 0]
    o_ref[...] = (
        jax.lax.dot_general(
            rel_ref[...], w_rel,
            dimension_numbers=(((2,), (0,)), ((), ())),
            preferred_element_type=jnp.float32,
        )
        + b_ref[0, 0]
    )


@jax.jit
def _rel_scores(relp3, W, b2):
    return pl.pallas_call(
        _score_body,
        out_shape=jax.ShapeDtypeStruct((8, 128), jnp.float32),
    )(relp3, W, b2)


def _sc_body(qrhi_hbm, qrlo_hbm, qo_hbm, scores_hbm, ent_hbm, wb_hbm,
             out_hbm,
             qrhi_v, qrlo_v, qo_v, scores_v, rows_v, out_v, wb_v,
             sem):
    wid = lax.axis_index("s") * NC + lax.axis_index("c")
    base = wid * BPW

    pltpu.sync_copy(qrhi_hbm.at[wid], qrhi_v)
    pltpu.sync_copy(qrlo_hbm.at[wid], qrlo_v)
    pltpu.sync_copy(qo_hbm.at[wid], qo_v)
    pltpu.sync_copy(scores_hbm, scores_v)
    pltpu.sync_copy(wb_hbm, wb_v)

    def issue(g, carry):
        qov = qo_v[pl.ds(g * LANES, LANES)]
        for lane in range(LANES):
            t = qov[lane]
            dst = rows_v.at[g * (LANES // 2) + lane // 2,
                            pl.ds((lane & 1) * EMB2, EMB2)]
            pltpu.async_copy(ent_hbm.at[t], dst, sem)
        return carry

    lax.fori_loop(0, BPW // LANES, issue, 0)
    pltpu.make_async_copy(ent_hbm.at[pl.ds(0, BPW // 2)], rows_v, sem).wait()

    w_vecs = [wb_v[pl.ds(k * LANES, LANES)] for k in range(8)]
    w_obj = [w_vecs[4 + d // LANES][d % LANES] for d in range(EMB2)]

    def group(g, carry):
        e_vec = lax.iota(jnp.int32, LANES) + g * LANES
        e_hi = e_vec >> 1
        e_lo = (e_vec & 1) * EMB2
        acc = plsc.load_gather(
            scores_v, [qrhi_v[pl.ds(g * LANES, LANES)],
                       qrlo_v[pl.ds(g * LANES, LANES)]])
        for d in range(EMB2):
            acc = acc + plsc.load_gather(rows_v, [e_hi, e_lo + d]) * w_obj[d]
        out_v[pl.ds(g * LANES, LANES)] = acc
        return carry

    lax.fori_loop(0, BPW // LANES, group, 0)
    pltpu.sync_copy(out_v, out_hbm.at[pl.ds(base, BPW)])


@jax.jit
def _classify(qr_hi, qr_lo, qo, scores, ent_table, wb):
    mesh = plsc.VectorSubcoreMesh(core_axis_name="c", subcore_axis_name="s")
    run = pl.kernel(
        _sc_body,
        out_type=jax.ShapeDtypeStruct((BATCH,), jnp.float32),
        mesh=mesh,
        compiler_params=pltpu.CompilerParams(needs_layout_passes=False),
        scratch_types=[
            pltpu.VMEM((BPW,), jnp.int32),
            pltpu.VMEM((BPW,), jnp.int32),
            pltpu.VMEM((BPW,), jnp.int32),
            pltpu.VMEM((8, 128), jnp.float32),
            pltpu.VMEM((BPW // 2, 128), jnp.float32),
            pltpu.VMEM((BPW,), jnp.float32),
            pltpu.VMEM((8 * LANES,), jnp.float32),
            pltpu.SemaphoreType.DMA,
        ],
    )
    return run(qr_hi, qr_lo, qo, scores, ent_table, wb)


def kernel(query_relation, query_object, relation_table, entity_table, W, b):
    qr = query_relation.astype(jnp.int32)
    qo = query_object.astype(jnp.int32)
    qr_hi = (qr >> 7).reshape(NW, BPW)
    qr_lo = (qr & 127).reshape(NW, BPW)
    relp3 = jnp.pad(relation_table, ((0, REL_PAD - 1000), (0, 0))).reshape(
        8, 128, EMB2)
    scores = _rel_scores(relp3, W, b.reshape(1, 1))
    wb = jnp.concatenate([W.reshape(4 * 32), jnp.zeros((0,), jnp.float32)])
    out = _classify(qr_hi, qr_lo, qo.reshape(NW, BPW), scores, entity_table,
                    wb)
    return out.reshape(BATCH, 1)


# speedup vs baseline: 2.7138x; 1.5840x over previous
import jax
import jax.numpy as jnp
from jax import lax
from jax.experimental import pallas as pl
from jax.experimental.pallas import tpu as pltpu
from jax.experimental.pallas import tpu_sc as plsc

BATCH = 16384
EMB2 = 64
NC, NS, LANES = 2, 16, 16
NW = NC * NS
BPW = BATCH // NW
CHUNK = 128
NCH = BPW // CHUNK
ENT_VOCAB = 1000000
EBLK = 4096
NEB = (ENT_VOCAB + EBLK - 1) // EBLK
ESROWS = NEB * (EBLK // 128)


def _rel_score_body(rel_ref, w_ref, b_ref, o_ref):
    w_rel = w_ref[0:EMB2, 0]
    o_ref[...] = (
        lax.dot_general(rel_ref[...], w_rel,
                        dimension_numbers=(((2,), (0,)), ((), ())),
                        preferred_element_type=jnp.float32)
        + b_ref[0, 0]
    )


@jax.jit
def _rel_scores(relp3, W, b2):
    return pl.pallas_call(
        _rel_score_body,
        out_shape=jax.ShapeDtypeStruct((8, 128), jnp.float32),
    )(relp3, W, b2)


def _ent_score_body(tnat_ref, w_ref, o_ref):
    w_obj = w_ref[EMB2:2 * EMB2, 0]
    x = tnat_ref[...].reshape(EMB2, EBLK // 128, 128)
    o_ref[...] = lax.dot_general(
        w_obj, x,
        dimension_numbers=(((0,), (0,)), ((), ())),
        preferred_element_type=jnp.float32)


@jax.jit
def _ent_scores(tnat, W):
    return pl.pallas_call(
        _ent_score_body,
        grid=(NEB,),
        in_specs=[
            pl.BlockSpec((EMB2, EBLK), lambda i: (0, i)),
            pl.BlockSpec((2 * EMB2, 1), lambda i: (0, 0)),
        ],
        out_specs=pl.BlockSpec((EBLK // 128, 128), lambda i: (i, 0)),
        out_shape=jax.ShapeDtypeStruct((ESROWS, 128), jnp.float32),
    )(tnat, W)


def _sc_body(qrhi_hbm, qrlo_hbm, qohi_hbm, qolo_hbm, rsc_hbm, esc_hbm,
             out_hbm,
             qrhi_v, qrlo_v, qohi_v, qolo_v, rsc_v, erows_v, out_v, sem):
    wid = lax.axis_index("s") * NC + lax.axis_index("c")
    base = wid * BPW

    pltpu.sync_copy(qrhi_hbm.at[wid], qrhi_v)
    pltpu.sync_copy(qrlo_hbm.at[wid], qrlo_v)
    pltpu.sync_copy(qohi_hbm.at[wid], qohi_v)
    pltpu.sync_copy(qolo_hbm.at[wid], qolo_v)
    pltpu.sync_copy(rsc_hbm, rsc_v)

    copies = [
        pltpu.async_copy(esc_hbm.at[qohi_v.at[c]],
                         erows_v.at[pl.ds(c * CHUNK, CHUNK)], sem)
        for c in range(NCH)
    ]
    for c in copies:
        c.wait()

    def group(g, carry):
        sl = pl.ds(g * LANES, LANES)
        e_vec = lax.iota(jnp.int32, LANES) + g * LANES
        acc = plsc.load_gather(rsc_v, [qrhi_v[sl], qrlo_v[sl]])
        acc = acc + plsc.load_gather(erows_v, [e_vec, qolo_v[sl]])
        out_v[sl] = acc
        return carry

    lax.fori_loop(0, BPW // LANES, group, 0)
    pltpu.sync_copy(out_v, out_hbm.at[pl.ds(base, BPW)])


@jax.jit
def _combine(qr_hi, qr_lo, qo_hi, qo_lo, rel_scores, ent_scores):
    mesh = plsc.VectorSubcoreMesh(core_axis_name="c", subcore_axis_name="s")
    run = pl.kernel(
        _sc_body,
        out_type=jax.ShapeDtypeStruct((BATCH,), jnp.float32),
        mesh=mesh,
        compiler_params=pltpu.CompilerParams(needs_layout_passes=False),
        scratch_types=[
            pltpu.VMEM((BPW,), jnp.int32),
            pltpu.VMEM((BPW,), jnp.int32),
            pltpu.VMEM((NCH, CHUNK), jnp.int32),
            pltpu.VMEM((BPW,), jnp.int32),
            pltpu.VMEM((8, 128), jnp.float32),
            pltpu.VMEM((BPW, 128), jnp.float32),
            pltpu.VMEM((BPW,), jnp.float32),
            pltpu.SemaphoreType.DMA,
        ],
    )
    return run(qr_hi, qr_lo, qo_hi, qo_lo, rel_scores, ent_scores)


def kernel(query_relation, query_object, relation_table, entity_table, W, b):
    qr = query_relation.astype(jnp.int32)
    qo = query_object.astype(jnp.int32)
    relp3 = jnp.pad(relation_table, ((0, 24), (0, 0))).reshape(8, 128, EMB2)
    rel_scores = _rel_scores(relp3, W, b.reshape(1, 1))
    ent_scores = _ent_scores(entity_table.T, W)
    out = _combine(
        (qr >> 7).reshape(NW, BPW), (qr & 127).reshape(NW, BPW),
        (qo >> 7).reshape(NW, NCH, CHUNK), (qo & 127).reshape(NW, BPW),
        rel_scores, ent_scores)
    return out.reshape(BATCH, 1)


# speedup vs baseline: 3.6076x; 1.3294x over previous
import jax
import jax.numpy as jnp
from jax import lax
from jax.experimental import pallas as pl
from jax.experimental.pallas import tpu as pltpu
from jax.experimental.pallas import tpu_sc as plsc

BATCH = 16384
EMB2 = 64
NC, NS, LANES = 2, 16, 16
NW = NC * NS
BPW = BATCH // NW
CHUNK = 128
NCH = BPW // CHUNK
ENT_VOCAB = 1000000
EBLK = 8192
NEB = (ENT_VOCAB + EBLK - 1) // EBLK
ESROWS = NEB * (EBLK // 128)


def _rel_score_body(rel_ref, w_ref, b_ref, o_ref):
    w_rel = w_ref[0:EMB2, 0]
    o_ref[...] = (
        lax.dot_general(rel_ref[...], w_rel,
                        dimension_numbers=(((2,), (0,)), ((), ())),
                        preferred_element_type=jnp.float32)
        + b_ref[0, 0]
    )


@jax.jit
def _rel_scores(relp3, W, b2):
    return pl.pallas_call(
        _rel_score_body,
        out_shape=jax.ShapeDtypeStruct((8, 128), jnp.float32),
    )(relp3, W, b2)


def _ent_score_body(tnat_ref, w_ref, o_ref):
    w_obj = w_ref[EMB2:2 * EMB2, 0]
    x = tnat_ref[...].reshape(EMB2, EBLK // 128, 128)
    o_ref[...] = lax.dot_general(
        w_obj, x,
        dimension_numbers=(((0,), (0,)), ((), ())),
        preferred_element_type=jnp.float32)


@jax.jit
def _ent_scores(tnat, W):
    return pl.pallas_call(
        _ent_score_body,
        grid=(NEB,),
        in_specs=[
            pl.BlockSpec((EMB2, EBLK), lambda i: (0, i)),
            pl.BlockSpec((2 * EMB2, 1), lambda i: (0, 0)),
        ],
        out_specs=pl.BlockSpec((EBLK // 128, 128), lambda i: (i, 0)),
        out_shape=jax.ShapeDtypeStruct((ESROWS, 128), jnp.float32),
    )(tnat, W)


def _sc_body(qrhi_hbm, qrlo_hbm, qohi_hbm, qolo_hbm, rsc_hbm, esc_hbm,
             out_hbm,
             qrhi_v, qrlo_v, qohi_v, qolo_v, rsc_v, erows_v, out_v, sem):
    wid = lax.axis_index("s") * NC + lax.axis_index("c")
    base = wid * BPW

    pltpu.sync_copy(qrhi_hbm.at[wid], qrhi_v)
    pltpu.sync_copy(qrlo_hbm.at[wid], qrlo_v)
    pltpu.sync_copy(qohi_hbm.at[wid], qohi_v)
    pltpu.sync_copy(qolo_hbm.at[wid], qolo_v)
    pltpu.sync_copy(rsc_hbm, rsc_v)

    copies = [
        pltpu.async_copy(esc_hbm.at[qohi_v.at[c]],
                         erows_v.at[pl.ds(c * CHUNK, CHUNK)], sem)
        for c in range(NCH)
    ]
    for c in copies:
        c.wait()

    def group(g, carry):
        sl = pl.ds(g * LANES, LANES)
        e_vec = lax.iota(jnp.int32, LANES) + g * LANES
        acc = plsc.load_gather(rsc_v, [qrhi_v[sl], qrlo_v[sl]])
        acc = acc + plsc.load_gather(erows_v, [e_vec, qolo_v[sl]])
        out_v[sl] = acc
        return carry

    lax.fori_loop(0, BPW // LANES, group, 0)
    pltpu.sync_copy(out_v, out_hbm.at[pl.ds(base, BPW)])


@jax.jit
def _combine(qr_hi, qr_lo, qo_hi, qo_lo, rel_scores, ent_scores):
    mesh = plsc.VectorSubcoreMesh(core_axis_name="c", subcore_axis_name="s")
    run = pl.kernel(
        _sc_body,
        out_type=jax.ShapeDtypeStruct((BATCH,), jnp.float32),
        mesh=mesh,
        compiler_params=pltpu.CompilerParams(needs_layout_passes=False),
        scratch_types=[
            pltpu.VMEM((BPW,), jnp.int32),
            pltpu.VMEM((BPW,), jnp.int32),
            pltpu.VMEM((NCH, CHUNK), jnp.int32),
            pltpu.VMEM((BPW,), jnp.int32),
            pltpu.VMEM((8, 128), jnp.float32),
            pltpu.VMEM((BPW, 128), jnp.float32),
            pltpu.VMEM((BPW,), jnp.float32),
            pltpu.SemaphoreType.DMA,
        ],
    )
    return run(qr_hi, qr_lo, qo_hi, qo_lo, rel_scores, ent_scores)


def kernel(query_relation, query_object, relation_table, entity_table, W, b):
    qr = query_relation.astype(jnp.int32)
    qo = query_object.astype(jnp.int32)
    relp3 = jnp.pad(relation_table, ((0, 24), (0, 0))).reshape(8, 128, EMB2)
    rel_scores = _rel_scores(relp3, W, b.reshape(1, 1))
    ent_scores = _ent_scores(entity_table.T, W)
    out = _combine(
        (qr >> 7).reshape(NW, BPW), (qr & 127).reshape(NW, BPW),
        (qo >> 7).reshape(NW, NCH, CHUNK), (qo & 127).reshape(NW, BPW),
        rel_scores, ent_scores)
    return out.reshape(BATCH, 1)


# speedup vs baseline: 5.0496x; 1.3997x over previous
import jax
import jax.numpy as jnp
from jax import lax
from jax.experimental import pallas as pl
from jax.experimental.pallas import tpu as pltpu
from jax.experimental.pallas import tpu_sc as plsc

BATCH = 16384
EMB2 = 64
NC, NS, LANES = 2, 16, 16
NW = NC * NS
BPW = BATCH // NW
CHUNK = 128
NCH = BPW // CHUNK
ENT_VOCAB = 1000000
EBLK = 32768
NEB = (ENT_VOCAB + EBLK - 1) // EBLK
ESROWS = NEB * (EBLK // 128)


def _rel_score_body(rel_ref, w_ref, b_ref, o_ref):
    w_rel = w_ref[0:EMB2, 0]
    o_ref[...] = (
        lax.dot_general(rel_ref[...], w_rel,
                        dimension_numbers=(((2,), (0,)), ((), ())),
                        preferred_element_type=jnp.float32)
        + b_ref[0, 0]
    )


@jax.jit
def _rel_scores(relp3, W, b2):
    return pl.pallas_call(
        _rel_score_body,
        out_shape=jax.ShapeDtypeStruct((8, 128), jnp.float32),
    )(relp3, W, b2)


def _ent_score_body(tnat_ref, w_ref, o_ref):
    w_obj = w_ref[EMB2:2 * EMB2, 0]
    x = tnat_ref[...].reshape(EMB2, EBLK // 128, 128)
    o_ref[...] = lax.dot_general(
        w_obj, x,
        dimension_numbers=(((0,), (0,)), ((), ())),
        preferred_element_type=jnp.float32)


@jax.jit
def _ent_scores(tnat, W):
    return pl.pallas_call(
        _ent_score_body,
        grid=(NEB,),
        in_specs=[
            pl.BlockSpec((EMB2, EBLK), lambda i: (0, i)),
            pl.BlockSpec((2 * EMB2, 1), lambda i: (0, 0)),
        ],
        out_specs=pl.BlockSpec((EBLK // 128, 128), lambda i: (i, 0)),
        out_shape=jax.ShapeDtypeStruct((ESROWS, 128), jnp.float32),
    )(tnat, W)


def _sc_body(qrhi_hbm, qrlo_hbm, qohi_hbm, qolo_hbm, rsc_hbm, esc_hbm,
             out_hbm,
             qrhi_v, qrlo_v, qohi_v, qolo_v, rsc_v, erows_v, out_v, sem):
    wid = lax.axis_index("s") * NC + lax.axis_index("c")
    base = wid * BPW

    pltpu.sync_copy(qrhi_hbm.at[wid], qrhi_v)
    pltpu.sync_copy(qrlo_hbm.at[wid], qrlo_v)
    pltpu.sync_copy(qohi_hbm.at[wid], qohi_v)
    pltpu.sync_copy(qolo_hbm.at[wid], qolo_v)
    pltpu.sync_copy(rsc_hbm, rsc_v)

    copies = [
        pltpu.async_copy(esc_hbm.at[qohi_v.at[c]],
                         erows_v.at[pl.ds(c * CHUNK, CHUNK)], sem)
        for c in range(NCH)
    ]
    for c in copies:
        c.wait()

    def group(g, carry):
        sl = pl.ds(g * LANES, LANES)
        e_vec = lax.iota(jnp.int32, LANES) + g * LANES
        acc = plsc.load_gather(rsc_v, [qrhi_v[sl], qrlo_v[sl]])
        acc = acc + plsc.load_gather(erows_v, [e_vec, qolo_v[sl]])
        out_v[sl] = acc
        return carry

    lax.fori_loop(0, BPW // LANES, group, 0)
    pltpu.sync_copy(out_v, out_hbm.at[pl.ds(base, BPW)])


@jax.jit
def _combine(qr_hi, qr_lo, qo_hi, qo_lo, rel_scores, ent_scores):
    mesh = plsc.VectorSubcoreMesh(core_axis_name="c", subcore_axis_name="s")
    run = pl.kernel(
        _sc_body,
        out_type=jax.ShapeDtypeStruct((BATCH,), jnp.float32),
        mesh=mesh,
        compiler_params=pltpu.CompilerParams(needs_layout_passes=False),
        scratch_types=[
            pltpu.VMEM((BPW,), jnp.int32),
            pltpu.VMEM((BPW,), jnp.int32),
            pltpu.VMEM((NCH, CHUNK), jnp.int32),
            pltpu.VMEM((BPW,), jnp.int32),
            pltpu.VMEM((8, 128), jnp.float32),
            pltpu.VMEM((BPW, 128), jnp.float32),
            pltpu.VMEM((BPW,), jnp.float32),
            pltpu.SemaphoreType.DMA,
        ],
    )
    return run(qr_hi, qr_lo, qo_hi, qo_lo, rel_scores, ent_scores)


def kernel(query_relation, query_object, relation_table, entity_table, W, b):
    qr = query_relation.astype(jnp.int32)
    qo = query_object.astype(jnp.int32)
    relp3 = jnp.pad(relation_table, ((0, 24), (0, 0))).reshape(8, 128, EMB2)
    rel_scores = _rel_scores(relp3, W, b.reshape(1, 1))
    ent_scores = _ent_scores(entity_table.T, W)
    out = _combine(
        (qr >> 7).reshape(NW, BPW), (qr & 127).reshape(NW, BPW),
        (qo >> 7).reshape(NW, NCH, CHUNK), (qo & 127).reshape(NW, BPW),
        rel_scores, ent_scores)
    return out.reshape(BATCH, 1)


# speedup vs baseline: 5.1820x; 1.0262x over previous
import jax
import jax.numpy as jnp
from jax import lax
from jax.experimental import pallas as pl
from jax.experimental.pallas import tpu as pltpu
from jax.experimental.pallas import tpu_sc as plsc

BATCH = 16384
EMB2 = 64
NC, NS, LANES = 2, 16, 16
NW = NC * NS
BPW = BATCH // NW
CHUNK = 128
NCH = BPW // CHUNK
ENT_VOCAB = 1000000
EBLK = 65536
NEB = (ENT_VOCAB + EBLK - 1) // EBLK
ESROWS = NEB * (EBLK // 128)


def _rel_score_body(rel_ref, w_ref, b_ref, o_ref):
    w_rel = w_ref[0:EMB2, 0]
    o_ref[...] = (
        lax.dot_general(rel_ref[...], w_rel,
                        dimension_numbers=(((2,), (0,)), ((), ())),
                        preferred_element_type=jnp.float32)
        + b_ref[0, 0]
    )


@jax.jit
def _rel_scores(relp3, W, b2):
    return pl.pallas_call(
        _rel_score_body,
        out_shape=jax.ShapeDtypeStruct((8, 128), jnp.float32),
    )(relp3, W, b2)


def _ent_score_body(tnat_ref, w_ref, o_ref):
    w_obj = w_ref[EMB2:2 * EMB2, 0]
    x = tnat_ref[...].reshape(EMB2, EBLK // 128, 128)
    o_ref[...] = lax.dot_general(
        w_obj, x,
        dimension_numbers=(((0,), (0,)), ((), ())),
        preferred_element_type=jnp.float32)


@jax.jit
def _ent_scores(tnat, W):
    return pl.pallas_call(
        _ent_score_body,
        grid=(NEB,),
        in_specs=[
            pl.BlockSpec((EMB2, EBLK), lambda i: (0, i)),
            pl.BlockSpec((2 * EMB2, 1), lambda i: (0, 0)),
        ],
        out_specs=pl.BlockSpec((EBLK // 128, 128), lambda i: (i, 0)),
        out_shape=jax.ShapeDtypeStruct((ESROWS, 128), jnp.float32),
    )(tnat, W)


def _sc_body(qrhi_hbm, qrlo_hbm, qohi_hbm, qolo_hbm, rsc_hbm, esc_hbm,
             out_hbm,
             qrhi_v, qrlo_v, qohi_v, qolo_v, rsc_v, erows_v, out_v, sem):
    wid = lax.axis_index("s") * NC + lax.axis_index("c")
    base = wid * BPW

    pltpu.sync_copy(qrhi_hbm.at[wid], qrhi_v)
    pltpu.sync_copy(qrlo_hbm.at[wid], qrlo_v)
    pltpu.sync_copy(qohi_hbm.at[wid], qohi_v)
    pltpu.sync_copy(qolo_hbm.at[wid], qolo_v)
    pltpu.sync_copy(rsc_hbm, rsc_v)

    copies = [
        pltpu.async_copy(esc_hbm.at[qohi_v.at[c]],
                         erows_v.at[pl.ds(c * CHUNK, CHUNK)], sem)
        for c in range(NCH)
    ]
    for c in copies:
        c.wait()

    def group(g, carry):
        sl = pl.ds(g * LANES, LANES)
        e_vec = lax.iota(jnp.int32, LANES) + g * LANES
        acc = plsc.load_gather(rsc_v, [qrhi_v[sl], qrlo_v[sl]])
        acc = acc + plsc.load_gather(erows_v, [e_vec, qolo_v[sl]])
        out_v[sl] = acc
        return carry

    lax.fori_loop(0, BPW // LANES, group, 0)
    pltpu.sync_copy(out_v, out_hbm.at[pl.ds(base, BPW)])


@jax.jit
def _combine(qr_hi, qr_lo, qo_hi, qo_lo, rel_scores, ent_scores):
    mesh = plsc.VectorSubcoreMesh(core_axis_name="c", subcore_axis_name="s")
    run = pl.kernel(
        _sc_body,
        out_type=jax.ShapeDtypeStruct((BATCH,), jnp.float32),
        mesh=mesh,
        compiler_params=pltpu.CompilerParams(needs_layout_passes=False),
        scratch_types=[
            pltpu.VMEM((BPW,), jnp.int32),
            pltpu.VMEM((BPW,), jnp.int32),
            pltpu.VMEM((NCH, CHUNK), jnp.int32),
            pltpu.VMEM((BPW,), jnp.int32),
            pltpu.VMEM((8, 128), jnp.float32),
            pltpu.VMEM((BPW, 128), jnp.float32),
            pltpu.VMEM((BPW,), jnp.float32),
            pltpu.SemaphoreType.DMA,
        ],
    )
    return run(qr_hi, qr_lo, qo_hi, qo_lo, rel_scores, ent_scores)


def kernel(query_relation, query_object, relation_table, entity_table, W, b):
    qr = query_relation.astype(jnp.int32)
    qo = query_object.astype(jnp.int32)
    relp3 = jnp.pad(relation_table, ((0, 24), (0, 0))).reshape(8, 128, EMB2)
    rel_scores = _rel_scores(relp3, W, b.reshape(1, 1))
    ent_scores = _ent_scores(entity_table.T, W)
    out = _combine(
        (qr >> 7).reshape(NW, BPW), (qr & 127).reshape(NW, BPW),
        (qo >> 7).reshape(NW, NCH, CHUNK), (qo & 127).reshape(NW, BPW),
        rel_scores, ent_scores)
    return out.reshape(BATCH, 1)
